# baseline (speedup 1.0000x reference)
import functools

import jax
import jax.numpy as jnp
from jax import lax
from jax.experimental import pallas as pl
from jax.experimental.pallas import tpu as pltpu
from jax.experimental.pallas import tpu_sc as plsc

NC = 2
NS = 16
NW = NC * NS
B = 128


def _round_up(a, m):
    return (a + m - 1) // m * m


def _hist_body(n_rows_tile, per_core, per_tile, rem,
               dst_hbm, ones_hbm, zeros_hbm, deg_out,
               i0, i1, i2, ones_v, zbuf,
               deg_sh, ih0, ih1, ih2, sh0, sh1):
    I = (i0, i1, i2)
    IH = (ih0, ih1, ih2)
    SH = (sh0, sh1)
    c = lax.axis_index("c")
    s = lax.axis_index("s")
    r0 = s * n_rows_tile
    pltpu.sync_copy(zeros_hbm, zbuf)
    pltpu.sync_copy(zbuf, deg_sh.at[pl.ds(r0, n_rows_tile)])
    pltpu.sync_copy(ones_hbm, ones_v)
    plsc.subcore_barrier()
    base = c * per_core + s * per_tile + jnp.minimum(s, rem)

    def idx_start(j):
        k = j % 3
        return pltpu.async_copy(dst_hbm.at[pl.ds((base + j) * B, B)],
                                I[k], IH[k])

    def scat_start(j):
        return pltpu.async_copy(ones_v, deg_sh.at[I[j % 3]], SH[j % 2],
                                add=True)

    ihd, shd = {}, {}
    ihd[0] = idx_start(0)
    if per_tile > 1:
        ihd[1] = idx_start(1)
    for j in range(per_tile):
        ihd[j].wait()
        shd[j] = scat_start(j)
        if j >= 1:
            shd[j - 1].wait()
        if j + 2 < per_tile:
            ihd[j + 2] = idx_start(j + 2)
    shd[per_tile - 1].wait()
    if rem:
        @pl.when(s < rem)
        def _extra():
            pltpu.sync_copy(dst_hbm.at[pl.ds((base + per_tile) * B, B)], i0)
            pltpu.sync_copy(ones_v, deg_sh.at[i0], add=True)
    plsc.subcore_barrier()
    nz = n_rows_tile * NS
    pltpu.sync_copy(deg_sh.at[pl.ds(r0, n_rows_tile)], zbuf)
    pltpu.sync_copy(zbuf, deg_out.at[pl.ds(c * nz + r0, n_rows_tile)])


def _row_chunks(n_rows_tile):
    offs, off = [], 0
    while off < n_rows_tile:
        sz = min(B, n_rows_tile - off)
        offs.append((off, sz))
        off += sz
    return offs


def _gs_body(n_rows_tile, per_core, per_tile, rem,
             h_hbm, src_hbm, dst_hbm, zeros_hbm, z_out,
             is0, is1, is2, id0, id1, id2, id3, rv0, rv1, rv2, z_sh,
             gs0, gs1, gs2, ss0, ss1, ixs0, ixs1, ixs2,
             ixd0, ixd1, ixd2, ixd3):
    S = (is0, is1, is2)
    D = (id0, id1, id2, id3)
    R = (rv0, rv1, rv2)
    GS = (gs0, gs1, gs2)
    SS = (ss0, ss1)
    IXS = (ixs0, ixs1, ixs2)
    IXD = (ixd0, ixd1, ixd2, ixd3)
    c = lax.axis_index("c")
    s = lax.axis_index("s")
    r0 = s * n_rows_tile
    pltpu.sync_copy(zeros_hbm, rv0)
    for off, sz in _row_chunks(n_rows_tile):
        pltpu.sync_copy(rv0.at[pl.ds(0, sz)], z_sh.at[pl.ds(r0 + off, sz)])
    plsc.subcore_barrier()
    base = c * per_core + s * per_tile + jnp.minimum(s, rem)

    def idx_s_start(j):
        return pltpu.async_copy(src_hbm.at[pl.ds((base + j) * B, B)],
                                S[j % 3], IXS[j % 3])

    def idx_d_start(j):
        return pltpu.async_copy(dst_hbm.at[pl.ds((base + j) * B, B)],
                                D[j % 4], IXD[j % 4])

    def gather_start(j):
        return pltpu.async_copy(h_hbm.at[S[j % 3]], R[j % 3], GS[j % 3])

    def scatter_start(j):
        return pltpu.async_copy(R[j % 3], z_sh.at[D[j % 4]], SS[j % 2],
                                add=True)

    sdd, gd, isd, idd = {}, {}, {}, {}
    for j in range(min(3, per_tile)):
        isd[j] = idx_s_start(j)
        idd[j] = idx_d_start(j)
    for j in range(min(2, per_tile)):
        isd[j].wait()
        gd[j] = gather_start(j)
    for j in range(per_tile):
        if j >= 1:
            sdd[j - 1].wait()
        gd[j].wait()
        if j + 3 < per_tile:
            isd[j + 3] = idx_s_start(j + 3)
            idd[j + 3] = idx_d_start(j + 3)
        if j + 2 < per_tile:
            isd[j + 2].wait()
            gd[j + 2] = gather_start(j + 2)
        idd[j].wait()
        sdd[j] = scatter_start(j)
    sdd[per_tile - 1].wait()
    if rem:
        @pl.when(s < rem)
        def _extra():
            b = (base + per_tile) * B
            pltpu.sync_copy(src_hbm.at[pl.ds(b, B)], is0)
            pltpu.sync_copy(dst_hbm.at[pl.ds(b, B)], id0)
            pltpu.async_copy(h_hbm.at[is0], rv0, gs0).wait()
            pltpu.sync_copy(rv0, z_sh.at[id0], add=True)
    plsc.subcore_barrier()
    for off, sz in _row_chunks(n_rows_tile):
        pltpu.sync_copy(z_sh.at[pl.ds(r0 + off, sz)], rv0.at[pl.ds(0, sz)])
        pltpu.sync_copy(rv0.at[pl.ds(0, sz)], z_out.at[c, pl.ds(r0 + off, sz)])


def _scale_mm_body(n, x_ref, w_ref, dp_ref, h_ref):
    deg = dp_ref[0, :n] + dp_ref[1, :n]
    r = lax.rsqrt(deg)
    h_ref[...] = jnp.dot(x_ref[...] * r, w_ref[...],
                         preferred_element_type=jnp.float32,
                         precision=lax.Precision.HIGHEST)


def _final_body(n, zp_ref, dp_ref, b_ref, o_ref):
    z = zp_ref[0, :n] + zp_ref[1, :n]
    deg = dp_ref[0, :n] + dp_ref[1, :n]
    r = lax.rsqrt(deg)
    o_ref[...] = r * z + deg * b_ref[...]


def kernel(x, edge_index, W, b):
    N, D_IN = x.shape
    D_OUT = W.shape[1]
    E = edge_index.shape[1]

    n_rows_tile = _round_up((N + NS - 1) // NS, 8)
    NZ = n_rows_tile * NS

    nb = E // B
    per_core = nb // NC
    per_tile = per_core // NS
    rem = per_core - per_tile * NS

    src = edge_index[0]
    dst = edge_index[1]

    ones = jnp.ones((B,), jnp.float32)
    zeros_d = jnp.zeros((n_rows_tile,), jnp.float32)
    zeros_z = jnp.zeros((B, D_OUT), jnp.float32)

    mesh = plsc.VectorSubcoreMesh(core_axis_name="c", subcore_axis_name="s")

    deg_part = pl.kernel(
        functools.partial(_hist_body, n_rows_tile, per_core, per_tile, rem),
        out_type=jax.ShapeDtypeStruct((NC * NZ,), jnp.float32),
        mesh=mesh,
        scratch_types=(
            [pltpu.VMEM((B,), jnp.int32)] * 3
            + [pltpu.VMEM((B,), jnp.float32)]
            + [pltpu.VMEM((n_rows_tile,), jnp.float32)]
            + [pltpu.VMEM_SHARED((NZ,), jnp.float32)]
            + [pltpu.SemaphoreType.DMA] * 5
        ),
    )(dst, ones, zeros_d)

    dp3 = deg_part.reshape(NC, NZ, 1)

    h = pl.pallas_call(
        functools.partial(_scale_mm_body, N),
        out_shape=jax.ShapeDtypeStruct((N, D_OUT), jnp.float32),
    )(x, W, dp3)

    z_part = pl.kernel(
        functools.partial(_gs_body, n_rows_tile, per_core, per_tile, rem),
        out_type=jax.ShapeDtypeStruct((NC, NZ, D_OUT), jnp.float32),
        mesh=mesh,
        scratch_types=(
            [pltpu.VMEM((B,), jnp.int32)] * 7
            + [pltpu.VMEM((B, D_OUT), jnp.float32)] * 3
            + [pltpu.VMEM_SHARED((NZ, D_OUT), jnp.float32)]
            + [pltpu.SemaphoreType.DMA] * 12
        ),
    )(h, src, dst, zeros_z)

    out = pl.pallas_call(
        functools.partial(_final_body, N),
        out_shape=jax.ShapeDtypeStruct((N, D_OUT), jnp.float32),
    )(z_part, dp3, b.reshape(1, D_OUT))

    return out


# speedup vs baseline: 1.0155x; 1.0155x over previous
import functools

import jax
import jax.numpy as jnp
from jax import lax
from jax.experimental import pallas as pl
from jax.experimental.pallas import tpu as pltpu
from jax.experimental.pallas import tpu_sc as plsc

NC = 2
NS = 16
NW = NC * NS
B = 128


def _round_up(a, m):
    return (a + m - 1) // m * m


def _hist_body(n_rows_tile, per_core, per_tile, rem,
               dst_hbm, ones_hbm, zeros_hbm, deg_out,
               i0, i1, i2, ones_v, zbuf,
               deg_sh, ih0, ih1, ih2, sh0, sh1):
    I = (i0, i1, i2)
    IH = (ih0, ih1, ih2)
    SH = (sh0, sh1)
    c = lax.axis_index("c")
    s = lax.axis_index("s")
    r0 = s * n_rows_tile
    pltpu.sync_copy(zeros_hbm, zbuf)
    pltpu.sync_copy(zbuf, deg_sh.at[pl.ds(r0, n_rows_tile)])
    pltpu.sync_copy(ones_hbm, ones_v)
    plsc.subcore_barrier()
    base = c * per_core + s * per_tile + jnp.minimum(s, rem)

    def idx_start(j):
        k = j % 3
        return pltpu.async_copy(dst_hbm.at[pl.ds((base + j) * B, B)],
                                I[k], IH[k])

    def scat_start(j):
        return pltpu.async_copy(ones_v, deg_sh.at[I[j % 3]], SH[j % 2],
                                add=True)

    ihd, shd = {}, {}
    ihd[0] = idx_start(0)
    if per_tile > 1:
        ihd[1] = idx_start(1)
    for j in range(per_tile):
        ihd[j].wait()
        shd[j] = scat_start(j)
        if j >= 1:
            shd[j - 1].wait()
        if j + 2 < per_tile:
            ihd[j + 2] = idx_start(j + 2)
    shd[per_tile - 1].wait()
    if rem:
        @pl.when(s < rem)
        def _extra():
            pltpu.sync_copy(dst_hbm.at[pl.ds((base + per_tile) * B, B)], i0)
            pltpu.sync_copy(ones_v, deg_sh.at[i0], add=True)
    plsc.subcore_barrier()
    nz = n_rows_tile * NS
    pltpu.sync_copy(deg_sh.at[pl.ds(r0, n_rows_tile)], zbuf)
    pltpu.sync_copy(zbuf, deg_out.at[pl.ds(c * nz + r0, n_rows_tile)])


def _row_chunks(n_rows_tile):
    offs, off = [], 0
    while off < n_rows_tile:
        sz = min(B, n_rows_tile - off)
        offs.append((off, sz))
        off += sz
    return offs


def _gs_body(n_rows_tile, per_core, per_tile, rem,
             h_hbm, src_hbm, dst_hbm, zeros_hbm, z_out,
             is0, is1, is2, id0, id1, id2, id3, rv0, rv1, rv2, z_sh,
             gs0, gs1, gs2, ss0, ss1, ixs0, ixs1, ixs2,
             ixd0, ixd1, ixd2, ixd3):
    S = (is0, is1, is2)
    D = (id0, id1, id2, id3)
    R = (rv0, rv1, rv2)
    GS = (gs0, gs1, gs2)
    SS = (ss0, ss1)
    IXS = (ixs0, ixs1, ixs2)
    IXD = (ixd0, ixd1, ixd2, ixd3)
    c = lax.axis_index("c")
    s = lax.axis_index("s")
    r0 = s * n_rows_tile
    pltpu.sync_copy(zeros_hbm, rv0)
    for off, sz in _row_chunks(n_rows_tile):
        pltpu.sync_copy(rv0.at[pl.ds(0, sz)], z_sh.at[pl.ds(r0 + off, sz)])
    plsc.subcore_barrier()
    base = c * per_core + s * per_tile + jnp.minimum(s, rem)

    def idx_s_start(j):
        return pltpu.async_copy(src_hbm.at[pl.ds((base + j) * B, B)],
                                S[j % 3], IXS[j % 3])

    def idx_d_start(j):
        return pltpu.async_copy(dst_hbm.at[pl.ds((base + j) * B, B)],
                                D[j % 4], IXD[j % 4])

    def gather_start(j):
        return pltpu.async_copy(h_hbm.at[S[j % 3]], R[j % 3], GS[j % 3])

    def scatter_start(j):
        return pltpu.async_copy(R[j % 3], z_sh.at[D[j % 4]], SS[j % 2],
                                add=True)

    sdd, gd, isd, idd = {}, {}, {}, {}
    for j in range(min(3, per_tile)):
        isd[j] = idx_s_start(j)
        idd[j] = idx_d_start(j)
    for j in range(min(2, per_tile)):
        isd[j].wait()
        gd[j] = gather_start(j)
    for j in range(per_tile):
        if j >= 1:
            sdd[j - 1].wait()
        gd[j].wait()
        if j + 3 < per_tile:
            isd[j + 3] = idx_s_start(j + 3)
            idd[j + 3] = idx_d_start(j + 3)
        if j + 2 < per_tile:
            isd[j + 2].wait()
            gd[j + 2] = gather_start(j + 2)
        idd[j].wait()
        sdd[j] = scatter_start(j)
    sdd[per_tile - 1].wait()
    if rem:
        @pl.when(s < rem)
        def _extra():
            b = (base + per_tile) * B
            pltpu.sync_copy(src_hbm.at[pl.ds(b, B)], is0)
            pltpu.sync_copy(dst_hbm.at[pl.ds(b, B)], id0)
            pltpu.async_copy(h_hbm.at[is0], rv0, gs0).wait()
            pltpu.sync_copy(rv0, z_sh.at[id0], add=True)
    plsc.subcore_barrier()
    for off, sz in _row_chunks(n_rows_tile):
        pltpu.sync_copy(z_sh.at[pl.ds(r0 + off, sz)], rv0.at[pl.ds(0, sz)])
        pltpu.sync_copy(rv0.at[pl.ds(0, sz)], z_out.at[c, pl.ds(r0 + off, sz)])


def _mm_body(x_ref, w_ref, m_ref):
    m_ref[...] = jnp.dot(x_ref[...], w_ref[...],
                         preferred_element_type=jnp.float32,
                         precision=lax.Precision.HIGHEST)


def _scale_body(n, m_ref, dp_ref, h_ref):
    deg = dp_ref[0, :n] + dp_ref[1, :n]
    r = lax.rsqrt(deg)
    h_ref[...] = m_ref[...] * r


def _final_body(n, zp_ref, dp_ref, b_ref, o_ref):
    z = zp_ref[0, :n] + zp_ref[1, :n]
    deg = dp_ref[0, :n] + dp_ref[1, :n]
    r = lax.rsqrt(deg)
    o_ref[...] = r * z + deg * b_ref[...]


def kernel(x, edge_index, W, b):
    N, D_IN = x.shape
    D_OUT = W.shape[1]
    E = edge_index.shape[1]

    n_rows_tile = _round_up((N + NS - 1) // NS, 8)
    NZ = n_rows_tile * NS

    nb = E // B
    per_core = nb // NC
    per_tile = per_core // NS
    rem = per_core - per_tile * NS

    src = edge_index[0]
    dst = edge_index[1]

    ones = jnp.ones((B,), jnp.float32)
    zeros_d = jnp.zeros((n_rows_tile,), jnp.float32)
    zeros_z = jnp.zeros((B, D_OUT), jnp.float32)

    mesh = plsc.VectorSubcoreMesh(core_axis_name="c", subcore_axis_name="s")

    deg_part = pl.kernel(
        functools.partial(_hist_body, n_rows_tile, per_core, per_tile, rem),
        out_type=jax.ShapeDtypeStruct((NC * NZ,), jnp.float32),
        mesh=mesh,
        scratch_types=(
            [pltpu.VMEM((B,), jnp.int32)] * 3
            + [pltpu.VMEM((B,), jnp.float32)]
            + [pltpu.VMEM((n_rows_tile,), jnp.float32)]
            + [pltpu.VMEM_SHARED((NZ,), jnp.float32)]
            + [pltpu.SemaphoreType.DMA] * 5
        ),
    )(dst, ones, zeros_d)

    dp3 = deg_part.reshape(NC, NZ, 1)

    m = pl.pallas_call(
        _mm_body,
        out_shape=jax.ShapeDtypeStruct((N, D_OUT), jnp.float32),
    )(x, W)

    h = pl.pallas_call(
        functools.partial(_scale_body, N),
        out_shape=jax.ShapeDtypeStruct((N, D_OUT), jnp.float32),
    )(m, dp3)

    z_part = pl.kernel(
        functools.partial(_gs_body, n_rows_tile, per_core, per_tile, rem),
        out_type=jax.ShapeDtypeStruct((NC, NZ, D_OUT), jnp.float32),
        mesh=mesh,
        scratch_types=(
            [pltpu.VMEM((B,), jnp.int32)] * 7
            + [pltpu.VMEM((B, D_OUT), jnp.float32)] * 3
            + [pltpu.VMEM_SHARED((NZ, D_OUT), jnp.float32)]
            + [pltpu.SemaphoreType.DMA] * 12
        ),
    )(h, src, dst, zeros_z)

    out = pl.pallas_call(
        functools.partial(_final_body, N),
        out_shape=jax.ShapeDtypeStruct((N, D_OUT), jnp.float32),
    )(z_part, dp3, b.reshape(1, D_OUT))

    return out
